# baseline (speedup 1.0000x reference)
import functools

import jax
import jax.numpy as jnp
import numpy as np
from jax import lax
from jax.experimental import pallas as pl
from jax.experimental.pallas import tpu as pltpu
from jax.experimental.pallas import tpu_sc as plsc

B = 16384
D = 128
NC = 2
NS = 16
NW = NC * NS
BPW = B // NW
CH = 128
NCH = BPW // CH
NSLOT = 2

_PERM = np.empty(D, np.int32)
for _c in range(D // 32):
    for _t in range(16):
        _PERM[32 * _c + 2 * _t] = 32 * _c + _t
        _PERM[32 * _c + 2 * _t + 1] = 32 * _c + 16 + _t

_mesh = plsc.VectorSubcoreMesh(core_axis_name="c", subcore_axis_name="s")


@functools.partial(
    pl.kernel,
    out_type=(
        jax.ShapeDtypeStruct((B, D), jnp.bfloat16),
        jax.ShapeDtypeStruct((B, D), jnp.bfloat16),
    ),
    mesh=_mesh,
    scratch_types=[
        pltpu.VMEM((NCH, CH), jnp.int32),
        pltpu.VMEM((NSLOT * CH, D), jnp.float32),
        pltpu.VMEM((BPW, D), jnp.bfloat16),
        pltpu.VMEM((BPW, D), jnp.bfloat16),
        pltpu.SemaphoreType.DMA((NSLOT,)),
        pltpu.SemaphoreType.DMA((2,)),
    ],
)
def _sc_gather(uid_hbm, iid_hbm, ut_hbm, it_hbm, uout_hbm, iout_hbm,
               idx_v, rows_v, bf_u, bf_i, gsem, osem):
    wid = lax.axis_index("s") * NC + lax.axis_index("c")
    base = wid * BPW
    out_copies = []
    for tbl, (ids_hbm, table_hbm, out_hbm, bf_v) in enumerate((
        (uid_hbm, ut_hbm, uout_hbm, bf_u),
        (iid_hbm, it_hbm, iout_hbm, bf_i),
    )):
        pltpu.sync_copy(ids_hbm.at[wid], idx_v)

        def gather(j):
            return pltpu.make_async_copy(
                table_hbm.at[idx_v.at[j]],
                rows_v.at[pl.ds((j % NSLOT) * CH, CH)],
                gsem.at[j % NSLOT],
            )

        for j in range(NSLOT):
            gather(j).start()
        for j in range(NCH):
            gather(j).wait()
            slot_base = (j % NSLOT) * CH
            dst_base = j * CH

            @functools.partial(plsc.parallel_loop, 0, CH * (D // 32),
                               unroll=8)
            def _convert(t):
                r = lax.shift_right_logical(t, 2)
                g = lax.bitwise_and(t, 3)
                a = rows_v[slot_base + r, pl.ds(g * 32, 16)]
                b = rows_v[slot_base + r, pl.ds(g * 32 + 16, 16)]
                bf_v[dst_base + r, pl.ds(g * 32, 32)] = plsc.pack(
                    a, b, format=plsc.PackFormat.INTERLEAVED)

            if j + NSLOT < NCH:
                gather(j + NSLOT).start()
        cp = pltpu.make_async_copy(
            bf_v, out_hbm.at[pl.ds(base, BPW)], osem.at[tbl])
        cp.start()
        out_copies.append(cp)
    for cp in out_copies:
        cp.wait()


BM = 16384


def _mlp_body(u_ref, i_ref, w1a_ref, w1b_ref, b1_ref, w2_ref, b2_ref,
              w3_ref, b3_ref, o_ref):
    u = u_ref[...].astype(jnp.float32)
    i = i_ref[...].astype(jnp.float32)
    h1 = jnp.dot(u, w1a_ref[...], preferred_element_type=jnp.float32)
    h1 += jnp.dot(i, w1b_ref[...], preferred_element_type=jnp.float32)
    h1 = jnp.maximum(h1 + b1_ref[...], 0.0)
    h2 = jnp.maximum(
        jnp.dot(h1, w2_ref[...], preferred_element_type=jnp.float32)
        + b2_ref[...], 0.0)
    o = jnp.maximum(
        jnp.dot(h2, w3_ref[...], preferred_element_type=jnp.float32)
        + b3_ref[...], 0.0)
    o_ref[...] = o


_mlp = pl.pallas_call(
    _mlp_body,
    grid=(B // BM,),
    in_specs=[
        pl.BlockSpec((BM, D), lambda i: (i, 0)),
        pl.BlockSpec((BM, D), lambda i: (i, 0)),
        pl.BlockSpec((D, 64), lambda i: (0, 0)),
        pl.BlockSpec((D, 64), lambda i: (0, 0)),
        pl.BlockSpec((1, 64), lambda i: (0, 0)),
        pl.BlockSpec((64, 32), lambda i: (0, 0)),
        pl.BlockSpec((1, 32), lambda i: (0, 0)),
        pl.BlockSpec((32, 1), lambda i: (0, 0)),
        pl.BlockSpec((1, 1), lambda i: (0, 0)),
    ],
    out_specs=pl.BlockSpec((BM, 1), lambda i: (i, 0)),
    out_shape=jax.ShapeDtypeStruct((B, 1), jnp.float32),
)


def kernel(user_id, item_id, user_table, item_table, W1, b1, W2, b2, W3, b3):
    uid = user_id.astype(jnp.int32).reshape(NW, NCH, CH)
    iid = item_id.astype(jnp.int32).reshape(NW, NCH, CH)
    u_emb, i_emb = _sc_gather(uid, iid, user_table, item_table)
    w1a = W1[:D][_PERM]
    w1b = W1[D:][_PERM]
    out = _mlp(u_emb, i_emb, w1a, w1b, b1.reshape(1, 64),
               W2, b2.reshape(1, 32), W3, b3.reshape(1, 1))
    return out.reshape(-1)


# speedup vs baseline: 1.0510x; 1.0510x over previous
import functools

import jax
import jax.numpy as jnp
import numpy as np
from jax import lax
from jax.experimental import pallas as pl
from jax.experimental.pallas import tpu as pltpu
from jax.experimental.pallas import tpu_sc as plsc

B = 16384
D = 128
NC = 2
NS = 16
NW = NC * NS
BPW = B // NW
CH = 128
NCH = BPW // CH
NSLOT = 2

_PERM = np.empty(D, np.int32)
for _c in range(D // 32):
    for _t in range(16):
        _PERM[32 * _c + 2 * _t] = 32 * _c + _t
        _PERM[32 * _c + 2 * _t + 1] = 32 * _c + 16 + _t
_PERM2 = np.concatenate([_PERM, D + _PERM])

_mesh = plsc.VectorSubcoreMesh(core_axis_name="c", subcore_axis_name="s")


@functools.partial(
    pl.kernel,
    out_type=jax.ShapeDtypeStruct((B, 2 * D), jnp.bfloat16),
    mesh=_mesh,
    scratch_types=[
        pltpu.VMEM((NCH, CH), jnp.int32),
        pltpu.VMEM((NSLOT * CH, D), jnp.float32),
        pltpu.VMEM((BPW, D), jnp.bfloat16),
        pltpu.VMEM((BPW, D), jnp.bfloat16),
        pltpu.SemaphoreType.DMA((NSLOT,)),
        pltpu.SemaphoreType.DMA((2,)),
    ],
)
def _sc_gather(uid_hbm, iid_hbm, ut_hbm, it_hbm, out_hbm,
               idx_v, rows_v, bf_u, bf_i, gsem, osem):
    wid = lax.axis_index("s") * NC + lax.axis_index("c")
    base = wid * BPW
    out_copies = []
    for tbl, (ids_hbm, table_hbm, bf_v) in enumerate((
        (uid_hbm, ut_hbm, bf_u),
        (iid_hbm, it_hbm, bf_i),
    )):
        pltpu.sync_copy(ids_hbm.at[wid], idx_v)

        def gather(j):
            return pltpu.make_async_copy(
                table_hbm.at[idx_v.at[j]],
                rows_v.at[pl.ds((j % NSLOT) * CH, CH)],
                gsem.at[j % NSLOT],
            )

        for j in range(NSLOT):
            gather(j).start()
        for j in range(NCH):
            gather(j).wait()
            slot_base = (j % NSLOT) * CH
            dst_base = j * CH

            @functools.partial(plsc.parallel_loop, 0, CH * (D // 32),
                               unroll=8)
            def _convert(t):
                r = lax.shift_right_logical(t, 2)
                g = lax.bitwise_and(t, 3)
                a = rows_v[slot_base + r, pl.ds(g * 32, 16)]
                b = rows_v[slot_base + r, pl.ds(g * 32 + 16, 16)]
                bf_v[dst_base + r, pl.ds(g * 32, 32)] = plsc.pack(
                    a, b, format=plsc.PackFormat.INTERLEAVED)

            if j + NSLOT < NCH:
                gather(j + NSLOT).start()
            cp = pltpu.make_async_copy(
                bf_v.at[pl.ds(dst_base, CH)],
                out_hbm.at[pl.ds(base + dst_base, CH), pl.ds(tbl * D, D)],
                osem.at[tbl],
            )
            cp.start()
            out_copies.append(cp)
    for cp in out_copies:
        cp.wait()


BM = 8192


def _mlp_body(x_ref, w1_ref, b1_ref, w2_ref, b2_ref, w3_ref, b3_ref, o_ref):
    x = x_ref[...].astype(jnp.float32)
    h1 = jnp.maximum(
        jnp.dot(x, w1_ref[...], preferred_element_type=jnp.float32)
        + b1_ref[...], 0.0)
    h2 = jnp.maximum(
        jnp.dot(h1, w2_ref[...], preferred_element_type=jnp.float32)
        + b2_ref[...], 0.0)
    o = jnp.maximum(
        jnp.dot(h2, w3_ref[...], preferred_element_type=jnp.float32)
        + b3_ref[...], 0.0)
    o_ref[...] = o


_mlp = pl.pallas_call(
    _mlp_body,
    grid=(B // BM,),
    in_specs=[
        pl.BlockSpec((BM, 2 * D), lambda i: (i, 0)),
        pl.BlockSpec((2 * D, 64), lambda i: (0, 0)),
        pl.BlockSpec((1, 64), lambda i: (0, 0)),
        pl.BlockSpec((64, 32), lambda i: (0, 0)),
        pl.BlockSpec((1, 32), lambda i: (0, 0)),
        pl.BlockSpec((32, 1), lambda i: (0, 0)),
        pl.BlockSpec((1, 1), lambda i: (0, 0)),
    ],
    out_specs=pl.BlockSpec((BM, 1), lambda i: (i, 0)),
    out_shape=jax.ShapeDtypeStruct((B, 1), jnp.float32),
)


def kernel(user_id, item_id, user_table, item_table, W1, b1, W2, b2, W3, b3):
    uid = user_id.astype(jnp.int32).reshape(NW, NCH, CH)
    iid = item_id.astype(jnp.int32).reshape(NW, NCH, CH)
    x = _sc_gather(uid, iid, user_table, item_table)
    out = _mlp(x, W1[_PERM2], b1.reshape(1, 64),
               W2, b2.reshape(1, 32), W3, b3.reshape(1, 1))
    return out.reshape(-1)


# speedup vs baseline: 1.0511x; 1.0001x over previous
import functools

import jax
import jax.numpy as jnp
import numpy as np
from jax import lax
from jax.experimental import pallas as pl
from jax.experimental.pallas import tpu as pltpu
from jax.experimental.pallas import tpu_sc as plsc

B = 16384
D = 128
NC = 2
NS = 16
NW = NC * NS
BPW = B // NW
CH = 128
NCH = BPW // CH
NSLOT = 2

_PERM = np.empty(D, np.int32)
for _c in range(D // 32):
    for _t in range(16):
        _PERM[32 * _c + 2 * _t] = 32 * _c + _t
        _PERM[32 * _c + 2 * _t + 1] = 32 * _c + 16 + _t
_PERM2 = np.concatenate([_PERM, D + _PERM])

_mesh = plsc.VectorSubcoreMesh(core_axis_name="c", subcore_axis_name="s")


@functools.partial(
    pl.kernel,
    out_type=jax.ShapeDtypeStruct((B, 2 * D), jnp.bfloat16),
    mesh=_mesh,
    scratch_types=[
        pltpu.VMEM((NCH, CH), jnp.int32),
        pltpu.VMEM((NSLOT * CH, D), jnp.float32),
        pltpu.VMEM((BPW, D), jnp.bfloat16),
        pltpu.VMEM((BPW, D), jnp.bfloat16),
        pltpu.SemaphoreType.DMA((NSLOT,)),
        pltpu.SemaphoreType.DMA((2,)),
    ],
)
def _sc_gather(uid_hbm, iid_hbm, ut_hbm, it_hbm, out_hbm,
               idx_v, rows_v, bf_u, bf_i, gsem, osem):
    wid = lax.axis_index("s") * NC + lax.axis_index("c")
    base = wid * BPW
    out_copies = []
    for tbl, (ids_hbm, table_hbm, bf_v) in enumerate((
        (uid_hbm, ut_hbm, bf_u),
        (iid_hbm, it_hbm, bf_i),
    )):
        pltpu.sync_copy(ids_hbm.at[wid], idx_v)

        def gather(j):
            return pltpu.make_async_copy(
                table_hbm.at[idx_v.at[j]],
                rows_v.at[pl.ds((j % NSLOT) * CH, CH)],
                gsem.at[j % NSLOT],
            )

        for j in range(NSLOT):
            gather(j).start()
        for j in range(NCH):
            gather(j).wait()
            slot_base = (j % NSLOT) * CH
            dst_base = j * CH

            @functools.partial(plsc.parallel_loop, 0, CH * (D // 32),
                               unroll=8)
            def _convert(t):
                r = lax.shift_right_logical(t, 2)
                g = lax.bitwise_and(t, 3)
                a = rows_v[slot_base + r, pl.ds(g * 32, 16)]
                b = rows_v[slot_base + r, pl.ds(g * 32 + 16, 16)]
                bf_v[dst_base + r, pl.ds(g * 32, 32)] = plsc.pack(
                    a, b, format=plsc.PackFormat.INTERLEAVED)

            if j + NSLOT < NCH:
                gather(j + NSLOT).start()
            cp = pltpu.make_async_copy(
                bf_v.at[pl.ds(dst_base, CH)],
                out_hbm.at[pl.ds(base + dst_base, CH), pl.ds(tbl * D, D)],
                osem.at[tbl],
            )
            cp.start()
            out_copies.append(cp)
    for cp in out_copies:
        cp.wait()


BM = 8192


def _mlp_body(x_ref, w1_ref, b1_ref, w2_ref, b2_ref, w3_ref, b3_ref, o_ref):
    h1 = jnp.maximum(
        jnp.dot(x_ref[...], w1_ref[...], preferred_element_type=jnp.float32)
        + b1_ref[...], 0.0)
    h2 = jnp.maximum(
        jnp.dot(h1, w2_ref[...], preferred_element_type=jnp.float32)
        + b2_ref[...], 0.0)
    o = jnp.maximum(
        jnp.dot(h2, w3_ref[...], preferred_element_type=jnp.float32)
        + b3_ref[...], 0.0)
    o_ref[...] = o


_mlp = pl.pallas_call(
    _mlp_body,
    grid=(B // BM,),
    in_specs=[
        pl.BlockSpec((BM, 2 * D), lambda i: (i, 0)),
        pl.BlockSpec((2 * D, 64), lambda i: (0, 0)),
        pl.BlockSpec((1, 64), lambda i: (0, 0)),
        pl.BlockSpec((64, 32), lambda i: (0, 0)),
        pl.BlockSpec((1, 32), lambda i: (0, 0)),
        pl.BlockSpec((32, 1), lambda i: (0, 0)),
        pl.BlockSpec((1, 1), lambda i: (0, 0)),
    ],
    out_specs=pl.BlockSpec((BM, 1), lambda i: (i, 0)),
    out_shape=jax.ShapeDtypeStruct((B, 1), jnp.float32),
)


def kernel(user_id, item_id, user_table, item_table, W1, b1, W2, b2, W3, b3):
    uid = user_id.astype(jnp.int32).reshape(NW, NCH, CH)
    iid = item_id.astype(jnp.int32).reshape(NW, NCH, CH)
    x = _sc_gather(uid, iid, user_table, item_table)
    out = _mlp(x, W1[_PERM2].astype(jnp.bfloat16), b1.reshape(1, 64),
               W2, b2.reshape(1, 32), W3, b3.reshape(1, 1))
    return out.reshape(-1)


# speedup vs baseline: 1.0517x; 1.0006x over previous
import functools

import jax
import jax.numpy as jnp
import numpy as np
from jax import lax
from jax.experimental import pallas as pl
from jax.experimental.pallas import tpu as pltpu
from jax.experimental.pallas import tpu_sc as plsc

B = 16384
D = 128
NC = 2
NS = 16
NW = NC * NS
BPW = B // NW
CH = 128
NCH = BPW // CH
NSLOT = 2

_PERM = np.empty(D, np.int32)
for _c in range(D // 32):
    for _t in range(16):
        _PERM[32 * _c + 2 * _t] = 32 * _c + _t
        _PERM[32 * _c + 2 * _t + 1] = 32 * _c + 16 + _t
_PERM2 = np.concatenate([_PERM, D + _PERM])

_mesh = plsc.VectorSubcoreMesh(core_axis_name="c", subcore_axis_name="s")


@functools.partial(
    pl.kernel,
    out_type=jax.ShapeDtypeStruct((B, 2 * D), jnp.bfloat16),
    mesh=_mesh,
    scratch_types=[
        pltpu.VMEM((2 * NCH, CH), jnp.int32),
        pltpu.VMEM((NSLOT * CH, D), jnp.float32),
        pltpu.VMEM((BPW, D), jnp.bfloat16),
        pltpu.VMEM((BPW, D), jnp.bfloat16),
        pltpu.SemaphoreType.DMA((NSLOT,)),
        pltpu.SemaphoreType.DMA((2,)),
    ],
)
def _sc_gather(uid_hbm, iid_hbm, ut_hbm, it_hbm, out_hbm,
               idx_v, rows_v, bf_u, bf_i, gsem, osem):
    wid = lax.axis_index("s") * NC + lax.axis_index("c")
    base = wid * BPW
    pltpu.sync_copy(uid_hbm.at[wid], idx_v.at[pl.ds(0, NCH)])
    pltpu.sync_copy(iid_hbm.at[wid], idx_v.at[pl.ds(NCH, NCH)])
    tables = (ut_hbm, it_hbm)
    bufs = (bf_u, bf_i)

    def gather(k):
        return pltpu.make_async_copy(
            tables[k // NCH].at[idx_v.at[k]],
            rows_v.at[pl.ds((k % NSLOT) * CH, CH)],
            gsem.at[k % NSLOT],
        )

    out_copies = []
    for k in range(NSLOT):
        gather(k).start()
    for k in range(2 * NCH):
        tbl, j = k // NCH, k % NCH
        gather(k).wait()
        slot_base = (k % NSLOT) * CH
        dst_base = j * CH
        bf_v = bufs[tbl]

        @functools.partial(plsc.parallel_loop, 0, CH * (D // 32),
                           unroll=8)
        def _convert(t):
            r = lax.shift_right_logical(t, 2)
            g = lax.bitwise_and(t, 3)
            a = rows_v[slot_base + r, pl.ds(g * 32, 16)]
            b = rows_v[slot_base + r, pl.ds(g * 32 + 16, 16)]
            bf_v[dst_base + r, pl.ds(g * 32, 32)] = plsc.pack(
                a, b, format=plsc.PackFormat.INTERLEAVED)

        if k + NSLOT < 2 * NCH:
            gather(k + NSLOT).start()
        cp = pltpu.make_async_copy(
            bf_v.at[pl.ds(dst_base, CH)],
            out_hbm.at[pl.ds(base + dst_base, CH), pl.ds(tbl * D, D)],
            osem.at[tbl],
        )
        cp.start()
        out_copies.append(cp)
    for cp in out_copies:
        cp.wait()


BM = 8192


def _mlp_body(x_ref, w1_ref, b1_ref, w2_ref, b2_ref, w3_ref, b3_ref, o_ref):
    h1 = jnp.maximum(
        jnp.dot(x_ref[...], w1_ref[...], preferred_element_type=jnp.float32)
        + b1_ref[...], 0.0)
    h2 = jnp.maximum(
        jnp.dot(h1, w2_ref[...], preferred_element_type=jnp.float32)
        + b2_ref[...], 0.0)
    o = jnp.maximum(
        jnp.dot(h2, w3_ref[...], preferred_element_type=jnp.float32)
        + b3_ref[...], 0.0)
    o_ref[...] = o


_mlp = pl.pallas_call(
    _mlp_body,
    grid=(B // BM,),
    in_specs=[
        pl.BlockSpec((BM, 2 * D), lambda i: (i, 0)),
        pl.BlockSpec((2 * D, 64), lambda i: (0, 0)),
        pl.BlockSpec((1, 64), lambda i: (0, 0)),
        pl.BlockSpec((64, 32), lambda i: (0, 0)),
        pl.BlockSpec((1, 32), lambda i: (0, 0)),
        pl.BlockSpec((32, 1), lambda i: (0, 0)),
        pl.BlockSpec((1, 1), lambda i: (0, 0)),
    ],
    out_specs=pl.BlockSpec((BM, 1), lambda i: (i, 0)),
    out_shape=jax.ShapeDtypeStruct((B, 1), jnp.float32),
)


def kernel(user_id, item_id, user_table, item_table, W1, b1, W2, b2, W3, b3):
    uid = user_id.astype(jnp.int32).reshape(NW, NCH, CH)
    iid = item_id.astype(jnp.int32).reshape(NW, NCH, CH)
    x = _sc_gather(uid, iid, user_table, item_table)
    out = _mlp(x, W1[_PERM2].astype(jnp.bfloat16), b1.reshape(1, 64),
               W2, b2.reshape(1, 32), W3, b3.reshape(1, 1))
    return out.reshape(-1)
